# baseline (speedup 1.0000x reference)
import functools

import jax
import jax.numpy as jnp
from jax import lax
from jax.experimental import pallas as pl
from jax.experimental.pallas import tpu as pltpu
from jax.experimental.pallas import tpu_sc as plsc

B = 16384
D = 64
NC = 2
NS = 16
NW = NC * NS
BPW = B // NW

_mesh = plsc.VectorSubcoreMesh(core_axis_name="c", subcore_axis_name="s")


@functools.partial(
    pl.kernel,
    mesh=_mesh,
    out_type=jax.ShapeDtypeStruct((B, D), jnp.float32),
    scratch_types=[
        pltpu.VMEM((BPW,), jnp.int32),
        pltpu.VMEM((BPW, D), jnp.float32),
        pltpu.SemaphoreType.DMA,
    ],
    compiler_params=pltpu.CompilerParams(use_tc_tiling_on_sc=False),
)
def _sc_gather(table_hbm, idx_hbm, out_hbm, idx_v, rows_v, sem):
    wid = lax.axis_index("s") * NC + lax.axis_index("c")
    base = wid * BPW
    pltpu.sync_copy(idx_hbm.at[pl.ds(base, BPW)], idx_v)
    pltpu.async_copy(table_hbm.at[idx_v], rows_v, sem).wait()
    pltpu.sync_copy(rows_v, out_hbm.at[pl.ds(base, BPW)])


def _blend_body(looked_ref, nv_ref, m_ref, w_ref, b_ref, out_ref):
    num = nv_ref[...] * w_ref[...] + b_ref[...]
    m = m_ref[...]
    out_ref[...] = m * num + (1.0 - m) * looked_ref[...]


_GRID = 8
_BLK = B // _GRID


def _blend(looked, nv, m, w, b):
    return pl.pallas_call(
        _blend_body,
        grid=(_GRID,),
        in_specs=[
            pl.BlockSpec((_BLK, D), lambda i: (i, 0)),
            pl.BlockSpec((_BLK, 1), lambda i: (i, 0)),
            pl.BlockSpec((_BLK, 1), lambda i: (i, 0)),
            pl.BlockSpec((1, D), lambda i: (0, 0)),
            pl.BlockSpec((1, D), lambda i: (0, 0)),
        ],
        out_specs=pl.BlockSpec((_BLK, D), lambda i: (i, 0)),
        out_shape=jax.ShapeDtypeStruct((B, D), jnp.float32),
    )(looked, nv, m, w, b)


def kernel(embedding_idx, numeric_value, is_numeric, table, W, b):
    idx = embedding_idx.astype(jnp.int32)
    looked = _sc_gather(table, idx)
    nv = numeric_value.reshape(B, 1)
    m = is_numeric.astype(jnp.float32).reshape(B, 1)
    w = W.reshape(1, D)
    bb = b.reshape(1, D)
    return _blend(looked, nv, m, w, bb)


# speedup vs baseline: 1.0438x; 1.0438x over previous
import functools

import jax
import jax.numpy as jnp
from jax import lax
from jax.experimental import pallas as pl
from jax.experimental.pallas import tpu as pltpu
from jax.experimental.pallas import tpu_sc as plsc

B = 16384
D = 64
NC = 2
BPC = B // NC

_smesh = plsc.ScalarSubcoreMesh(axis_name="c")


@functools.partial(
    pl.kernel,
    mesh=_smesh,
    out_type=jax.ShapeDtypeStruct((B, D), jnp.float32),
    scratch_types=[
        pltpu.SMEM((BPC,), jnp.int32),
        pltpu.SemaphoreType.DMA,
        pltpu.SemaphoreType.DMA,
    ],
)
def _sc_gather(table_hbm, idx_hbm, out_hbm, idx_s, sem_i, sem_g):
    cid = lax.axis_index("c")
    base = cid * BPC
    pltpu.async_copy(idx_hbm.at[pl.ds(base, BPC)], idx_s, sem_i).wait()

    @pl.loop(0, BPC)
    def _issue(i):
        r = idx_s[i]
        pltpu.async_copy(
            table_hbm.at[pl.ds(r, 1)], out_hbm.at[pl.ds(base + i, 1)], sem_g
        )

    pltpu.make_async_copy(
        table_hbm.at[pl.ds(0, BPC)], out_hbm.at[pl.ds(base, BPC)], sem_g
    ).wait()


def _blend_body(looked_ref, nv_ref, m_ref, w_ref, b_ref, out_ref):
    num = nv_ref[...] * w_ref[...] + b_ref[...]
    m = m_ref[...]
    out_ref[...] = m * num + (1.0 - m) * looked_ref[...]


_GRID = 8
_BLK = B // _GRID


def _blend(looked, nv, m, w, b):
    return pl.pallas_call(
        _blend_body,
        grid=(_GRID,),
        in_specs=[
            pl.BlockSpec((_BLK, D), lambda i: (i, 0)),
            pl.BlockSpec((_BLK, 1), lambda i: (i, 0)),
            pl.BlockSpec((_BLK, 1), lambda i: (i, 0)),
            pl.BlockSpec((1, D), lambda i: (0, 0)),
            pl.BlockSpec((1, D), lambda i: (0, 0)),
        ],
        out_specs=pl.BlockSpec((_BLK, D), lambda i: (i, 0)),
        out_shape=jax.ShapeDtypeStruct((B, D), jnp.float32),
    )(looked, nv, m, w, b)


def kernel(embedding_idx, numeric_value, is_numeric, table, W, b):
    idx = embedding_idx.astype(jnp.int32)
    looked = _sc_gather(table, idx)
    nv = numeric_value.reshape(B, 1)
    m = is_numeric.astype(jnp.float32).reshape(B, 1)
    w = W.reshape(1, D)
    bb = b.reshape(1, D)
    return _blend(looked, nv, m, w, bb)
